# baseline (speedup 1.0000x reference)
import jax
import jax.numpy as jnp
from jax import lax
from jax.experimental import pallas as pl
from jax.experimental.pallas import tpu as pltpu
from jax.experimental.pallas import tpu_sc as plsc

_NC = 2
_NS = 16
_NW = _NC * _NS
_L = 16
_CH = 128


def _sc_body(theta_hbm, pidx_hbm, cidx_hbm, uidx_hbm, duv_hbm, duw_hbm,
             pidx_v, cidx_v, uidx_v, u_rows, v_rows, w_rows,
             duv_v, duw_v, sem):
    D = theta_hbm.shape[1]
    n_chunks = pidx_v.shape[0]
    bpw = n_chunks * _CH
    wid = lax.axis_index("s") * _NC + lax.axis_index("c")
    base_chunk = wid * n_chunks

    pltpu.sync_copy(pidx_hbm.at[pl.ds(base_chunk, n_chunks)], pidx_v)
    pltpu.sync_copy(cidx_hbm.at[pl.ds(base_chunk, n_chunks)], cidx_v)
    pltpu.sync_copy(uidx_hbm.at[pl.ds(base_chunk, n_chunks)], uidx_v)

    descs = []
    for j in range(n_chunks):
        dst = pl.ds(j * _CH, _CH)
        descs.append(pltpu.async_copy(theta_hbm.at[pidx_v.at[j]], u_rows.at[dst], sem))
        descs.append(pltpu.async_copy(theta_hbm.at[cidx_v.at[j]], v_rows.at[dst], sem))
        descs.append(pltpu.async_copy(theta_hbm.at[uidx_v.at[j]], w_rows.at[dst], sem))
    for dsc in descs:
        dsc.wait()

    lane = lax.iota(jnp.int32, _L)
    for g in range(bpw // _L):
        row = jnp.full((_L,), g * _L, jnp.int32) + lane
        col0 = jnp.zeros((_L,), jnp.int32)
        u0 = plsc.load_gather(u_rows, [row, col0])
        v0 = plsc.load_gather(v_rows, [row, col0])
        w0 = plsc.load_gather(w_rows, [row, col0])

        def body(dcol, accs, row=row):
            a_uv, a_uw = accs
            colv = jnp.full((_L,), dcol, jnp.int32)
            ud = plsc.load_gather(u_rows, [row, colv])
            vd = plsc.load_gather(v_rows, [row, colv])
            wd = plsc.load_gather(w_rows, [row, colv])
            return (a_uv - ud * vd, a_uw - ud * wd)

        acc_uv, acc_uw = lax.fori_loop(1, D, body, (u0 * v0, u0 * w0))
        duv_v[pl.ds(g * _L, _L)] = acc_uv
        duw_v[pl.ds(g * _L, _L)] = acc_uw

    pltpu.sync_copy(duv_v, duv_hbm.at[pl.ds(wid * bpw, bpw)])
    pltpu.sync_copy(duw_v, duw_hbm.at[pl.ds(wid * bpw, bpw)])


def _acosh_body(duv_ref, duw_ref, ouv_ref, ouw_ref):
    for s, o in ((duv_ref, ouv_ref), (duw_ref, ouw_ref)):
        d = jnp.maximum(s[...], 1.0 + 1e-07)
        o[...] = jnp.log(d + jnp.sqrt(d * d - 1.0))


def kernel(theta, parent, child, unrelated):
    B = parent.shape[0]
    D = theta.shape[1]
    bpw = B // _NW
    n_chunks = bpw // _CH
    idx2 = (B // _CH, _CH)
    p2 = parent.astype(jnp.int32).reshape(idx2)
    c2 = child.astype(jnp.int32).reshape(idx2)
    u2 = unrelated.astype(jnp.int32).reshape(idx2)

    sc = pl.kernel(
        _sc_body,
        out_type=(jax.ShapeDtypeStruct((B,), jnp.float32),
                  jax.ShapeDtypeStruct((B,), jnp.float32)),
        mesh=plsc.VectorSubcoreMesh(core_axis_name="c", subcore_axis_name="s",
                                    num_cores=_NC, num_subcores=_NS),
        compiler_params=pltpu.CompilerParams(needs_layout_passes=False,
                                             use_tc_tiling_on_sc=False),
        scratch_types=[
            pltpu.VMEM((n_chunks, _CH), jnp.int32),
            pltpu.VMEM((n_chunks, _CH), jnp.int32),
            pltpu.VMEM((n_chunks, _CH), jnp.int32),
            pltpu.VMEM((bpw, D), jnp.float32),
            pltpu.VMEM((bpw, D), jnp.float32),
            pltpu.VMEM((bpw, D), jnp.float32),
            pltpu.VMEM((bpw,), jnp.float32),
            pltpu.VMEM((bpw,), jnp.float32),
            pltpu.SemaphoreType.DMA,
        ],
    )
    duv, duw = sc(theta, p2, c2, u2)

    tc = pl.pallas_call(
        _acosh_body,
        out_shape=(jax.ShapeDtypeStruct((B // _CH, _CH), jnp.float32),
                   jax.ShapeDtypeStruct((B // _CH, _CH), jnp.float32)),
    )
    ouv, ouw = tc(duv.reshape(B // _CH, _CH), duw.reshape(B // _CH, _CH))
    return ouv.reshape(B), ouw.reshape(B)
